# baseline (speedup 1.0000x reference)
import functools

import jax
import jax.numpy as jnp
from jax import lax
from jax.experimental import pallas as pl
from jax.experimental.pallas import tpu as pltpu
from jax.experimental.pallas import tpu_sc as plsc

MAXLEN = 200
EMBED = 32
VOCAB = 1000000
BATCH = 4096
NC = 2
NS = 16
NW = NC * NS
LANES = 16

BLK = 80
NBLK = VOCAB // BLK
NBMAX = -(-NBLK // NS)
ROWS_PER_W = BATCH // NW
SEG = 40
NSEG = MAXLEN // SEG
NQ = ROWS_PER_W * NSEG


def _make_kernel():
  mesh = plsc.VectorSubcoreMesh(core_axis_name="c", subcore_axis_name="s")

  @functools.partial(
      pl.kernel,
      mesh=mesh,
      out_type=jax.ShapeDtypeStruct((BATCH, MAXLEN, EMBED), jnp.float32),
      scratch_types=[
          pltpu.HBM((VOCAB, 128), jnp.float32),
          pltpu.VMEM((2, BLK, EMBED), jnp.float32),
          pltpu.VMEM((2, BLK, 128), jnp.float32),
          pltpu.VMEM((2, SEG, 128), jnp.float32),
          pltpu.VMEM((2, SEG, EMBED), jnp.float32),
          pltpu.VMEM((SEG,), jnp.int32),
          pltpu.VMEM((SEG,), jnp.int32),
          pltpu.VMEM((MAXLEN, EMBED), jnp.float32),
          pltpu.SemaphoreType.DMA,
          pltpu.SemaphoreType.DMA,
          pltpu.SemaphoreType.DMA,
          pltpu.SemaphoreType.DMA,
      ],
  )
  def emb_kernel(x_hbm, tok_hbm, pos_hbm, out_hbm,
                 scr, stage_v, wide_v, g_v, o_v, idx0_v, idx1_v, pos_v,
                 sem0, sem1, sem2, sem3):
    cid = lax.axis_index("c")
    sid = lax.axis_index("s")
    wid = sid * NC + cid
    isems = (sem0, sem1)
    osems = (sem2, sem3)

    pltpu.sync_copy(pos_hbm, pos_v)

    def blk_in(k, b, sem):
      pltpu.async_copy(
          tok_hbm.at[pl.ds((sid + NS * k) * BLK, BLK)], stage_v.at[b], sem)

    def blk_out(k, b, sem):
      pltpu.async_copy(
          wide_v.at[b, :, pl.ds(0, EMBED)],
          scr.at[pl.ds((sid + NS * k) * BLK, BLK), pl.ds(0, EMBED)], sem)

    def blk_exists(k):
      return sid + NS * k < NBLK

    @pl.when(blk_exists(0))
    def _prime():
      blk_in(0, 0, isems[0])

    @pl.loop(0, NBMAX, step=2)
    def blk_loop(k0):
      for b in range(2):
        nb = 1 - b
        k = k0 + b

        @pl.when(blk_exists(k))
        def _run_blk():
          @pl.when(blk_exists(k + 1))
          def _issue_next_in():
            @pl.when(k >= 1)
            def _drain_prev_out():
              pltpu.make_async_copy(
                  wide_v.at[nb, :, pl.ds(0, EMBED)],
                  scr.at[pl.ds(0, BLK), pl.ds(0, EMBED)],
                  osems[nb]).wait()
            blk_in(k + 1, nb, isems[nb])

          pltpu.make_async_copy(
              tok_hbm.at[pl.ds(0, BLK)], stage_v.at[b], isems[b]).wait()

          @pl.loop(0, BLK, unroll=8)
          def r_body(r):
            for h in range(EMBED // LANES):
              wide_v[b, r, pl.ds(h * LANES, LANES)] = (
                  stage_v[b, r, pl.ds(h * LANES, LANES)])

          blk_out(k, b, osems[b])

    @pl.when(blk_exists(0))
    def _drain_tail():
      pltpu.make_async_copy(
          wide_v.at[0, :, pl.ds(0, EMBED)],
          scr.at[pl.ds(0, BLK), pl.ds(0, EMBED)], osems[0]).wait()

      @pl.when(blk_exists(1))
      def _drain_tail1():
        pltpu.make_async_copy(
            wide_v.at[1, :, pl.ds(0, EMBED)],
            scr.at[pl.ds(0, BLK), pl.ds(0, EMBED)], osems[1]).wait()

    plsc.subcore_barrier()

    row0 = wid * ROWS_PER_W
    idxs = (idx0_v, idx1_v)

    def fetch(q, b, sem):
      pltpu.sync_copy(
          x_hbm.at[pl.ds(row0 * MAXLEN + q * SEG, SEG)], idxs[b])
      pltpu.async_copy(scr.at[idxs[b]], g_v.at[b], sem)

    def out_ref(q):
      return out_hbm.at[row0 + q // NSEG, pl.ds((q % NSEG) * SEG, SEG)]

    fetch(0, 0, isems[0])

    @pl.loop(0, NQ, step=2)
    def seg_loop(q0):
      for b in range(2):
        nb = 1 - b
        q = q0 + b

        @pl.when(q + 1 < NQ)
        def _issue_next():
          @pl.when(q >= 1)
          def _drain_prev_out():
            pltpu.make_async_copy(o_v.at[nb], out_ref(0), osems[nb]).wait()
          fetch(q + 1, nb, isems[nb])

        pltpu.make_async_copy(scr.at[idxs[b]], g_v.at[b], isems[b]).wait()
        tpos = (q % NSEG) * SEG

        @pl.loop(0, SEG, unroll=4)
        def t_body(t):
          for h in range(EMBED // LANES):
            o_v[b, t, pl.ds(h * LANES, LANES)] = (
                g_v[b, t, pl.ds(h * LANES, LANES)]
                + pos_v[tpos + t, pl.ds(h * LANES, LANES)])

        pltpu.async_copy(o_v.at[b], out_ref(q), osems[b])

    pltpu.make_async_copy(o_v.at[0], out_ref(0), osems[NQ % 2]).wait()
    pltpu.make_async_copy(o_v.at[1], out_ref(0), osems[(NQ + 1) % 2]).wait()

  return emb_kernel


def kernel(x, token_table, pos_table):
  x_flat = x.reshape(-1).astype(jnp.int32)
  return _make_kernel()(x_flat, token_table, pos_table)


# speedup vs baseline: 1.0857x; 1.0857x over previous
import functools

import jax
import jax.numpy as jnp
from jax import lax
from jax.experimental import pallas as pl
from jax.experimental.pallas import tpu as pltpu
from jax.experimental.pallas import tpu_sc as plsc

MAXLEN = 200
EMBED = 32
VOCAB = 1000000
BATCH = 4096
NC = 2
NS = 16
NW = NC * NS
LANES = 16

BLK = 80
NBLK = VOCAB // BLK
NBMAX = -(-NBLK // NS)
ROWS_PER_W = BATCH // NW
SEG = 40
NSEG = MAXLEN // SEG
NQ = ROWS_PER_W * NSEG


def _make_kernel():
  mesh = plsc.VectorSubcoreMesh(core_axis_name="c", subcore_axis_name="s")

  @functools.partial(
      pl.kernel,
      mesh=mesh,
      out_type=jax.ShapeDtypeStruct((BATCH, MAXLEN, EMBED), jnp.float32),
      scratch_types=[
          pltpu.HBM((VOCAB, 128), jnp.float32),
          pltpu.VMEM((2, BLK, EMBED), jnp.float32),
          pltpu.VMEM((2, BLK, 128), jnp.float32),
          pltpu.VMEM((2, SEG, 128), jnp.float32),
          pltpu.VMEM((2, SEG, EMBED), jnp.float32),
          pltpu.VMEM((ROWS_PER_W * MAXLEN,), jnp.int32),
          pltpu.VMEM((MAXLEN, EMBED), jnp.float32),
          pltpu.SemaphoreType.DMA,
          pltpu.SemaphoreType.DMA,
          pltpu.SemaphoreType.DMA,
          pltpu.SemaphoreType.DMA,
      ],
  )
  def emb_kernel(x_hbm, tok_hbm, pos_hbm, out_hbm,
                 scr, stage_v, wide_v, g_v, o_v, idx_v, pos_v,
                 sem0, sem1, sem2, sem3):
    cid = lax.axis_index("c")
    sid = lax.axis_index("s")
    wid = sid * NC + cid
    isems = (sem0, sem1)
    osems = (sem2, sem3)

    pltpu.sync_copy(pos_hbm, pos_v)

    def blk_in(k, b, sem):
      pltpu.async_copy(
          tok_hbm.at[pl.ds((sid + NS * k) * BLK, BLK)], stage_v.at[b], sem)

    def blk_out(k, b, sem):
      pltpu.async_copy(
          wide_v.at[b, :, pl.ds(0, EMBED)],
          scr.at[pl.ds((sid + NS * k) * BLK, BLK), pl.ds(0, EMBED)], sem)

    def blk_exists(k):
      return sid + NS * k < NBLK

    @pl.when(blk_exists(0))
    def _prime():
      blk_in(0, 0, isems[0])

    @pl.loop(0, NBMAX, step=2)
    def blk_loop(k0):
      for b in range(2):
        nb = 1 - b
        k = k0 + b

        @pl.when(blk_exists(k))
        def _run_blk():
          @pl.when(blk_exists(k + 1))
          def _issue_next_in():
            @pl.when(k >= 1)
            def _drain_prev_out():
              pltpu.make_async_copy(
                  wide_v.at[nb, :, pl.ds(0, EMBED)],
                  scr.at[pl.ds(0, BLK), pl.ds(0, EMBED)],
                  osems[nb]).wait()
            blk_in(k + 1, nb, isems[nb])

          pltpu.make_async_copy(
              tok_hbm.at[pl.ds(0, BLK)], stage_v.at[b], isems[b]).wait()

          @pl.loop(0, BLK, unroll=8)
          def r_body(r):
            for h in range(EMBED // LANES):
              wide_v[b, r, pl.ds(h * LANES, LANES)] = (
                  stage_v[b, r, pl.ds(h * LANES, LANES)])

          blk_out(k, b, osems[b])

    @pl.when(blk_exists(0))
    def _drain_tail():
      pltpu.make_async_copy(
          wide_v.at[0, :, pl.ds(0, EMBED)],
          scr.at[pl.ds(0, BLK), pl.ds(0, EMBED)], osems[0]).wait()

      @pl.when(blk_exists(1))
      def _drain_tail1():
        pltpu.make_async_copy(
            wide_v.at[1, :, pl.ds(0, EMBED)],
            scr.at[pl.ds(0, BLK), pl.ds(0, EMBED)], osems[1]).wait()

    plsc.subcore_barrier()

    row0 = wid * ROWS_PER_W
    pltpu.sync_copy(
        x_hbm.at[pl.ds(row0 * MAXLEN, ROWS_PER_W * MAXLEN)], idx_v)

    def fetch(q, b, sem):
      pltpu.async_copy(
          scr.at[idx_v.at[pl.ds(q * SEG, SEG)]], g_v.at[b], sem)

    def out_ref(q):
      return out_hbm.at[row0 + q // NSEG, pl.ds((q % NSEG) * SEG, SEG)]

    fetch(0, 0, isems[0])

    @pl.loop(0, NQ, step=2)
    def seg_loop(q0):
      for b in range(2):
        nb = 1 - b
        q = q0 + b

        @pl.when(q + 1 < NQ)
        def _issue_next():
          @pl.when(q >= 1)
          def _drain_prev_out():
            pltpu.make_async_copy(o_v.at[nb], out_ref(0), osems[nb]).wait()
          fetch(q + 1, nb, isems[nb])

        pltpu.make_async_copy(
            scr.at[idx_v.at[pl.ds(0, SEG)]], g_v.at[b], isems[b]).wait()
        tpos = (q % NSEG) * SEG

        @pl.loop(0, SEG, unroll=4)
        def t_body(t):
          for h in range(EMBED // LANES):
            o_v[b, t, pl.ds(h * LANES, LANES)] = (
                g_v[b, t, pl.ds(h * LANES, LANES)]
                + pos_v[tpos + t, pl.ds(h * LANES, LANES)])

        pltpu.async_copy(o_v.at[b], out_ref(q), osems[b])

    pltpu.make_async_copy(o_v.at[0], out_ref(0), osems[NQ % 2]).wait()
    pltpu.make_async_copy(o_v.at[1], out_ref(0), osems[(NQ + 1) % 2]).wait()

  return emb_kernel


def kernel(x, token_table, pos_table):
  x_flat = x.reshape(-1).astype(jnp.int32)
  return _make_kernel()(x_flat, token_table, pos_table)


# speedup vs baseline: 2.1719x; 2.0005x over previous
import functools

import jax
import jax.numpy as jnp
from jax import lax
from jax.experimental import pallas as pl
from jax.experimental.pallas import tpu as pltpu
from jax.experimental.pallas import tpu_sc as plsc

MAXLEN = 200
EMBED = 32
NC = 2
NS = 16
NW = NC * NS
LANES = 16

CHUNK = 800
REPS = CHUNK // MAXLEN


def _make_kernel(total_rows: int):
  rows_per_w = total_rows // NW
  nchunks = rows_per_w // CHUNK
  mesh = plsc.VectorSubcoreMesh(core_axis_name="c", subcore_axis_name="s")

  @functools.partial(
      pl.kernel,
      mesh=mesh,
      out_type=jax.ShapeDtypeStruct((total_rows, EMBED), jnp.float32),
      compiler_params=pltpu.CompilerParams(use_tc_tiling_on_sc=False),
      scratch_types=[
          pltpu.VMEM((rows_per_w,), jnp.int32),
          pltpu.VMEM((2, CHUNK, EMBED), jnp.float32),
          pltpu.VMEM((MAXLEN, EMBED), jnp.float32),
          pltpu.SemaphoreType.DMA,
          pltpu.SemaphoreType.DMA,
          pltpu.SemaphoreType.DMA,
          pltpu.SemaphoreType.DMA,
      ],
  )
  def emb_kernel(x_hbm, tok_hbm, pos_hbm, out_hbm,
                 idx_v, rows_v, pos_v, gsem0, gsem1, osem0, osem1):
    wid = lax.axis_index("s") * NC + lax.axis_index("c")
    base0 = wid * rows_per_w
    gsems = (gsem0, gsem1)
    osems = (osem0, osem1)

    pltpu.sync_copy(pos_hbm, pos_v)
    pltpu.sync_copy(x_hbm.at[pl.ds(base0, rows_per_w)], idx_v)

    def gather(ci, b, sem):
      pltpu.async_copy(
          tok_hbm.at[idx_v.at[pl.ds(ci * CHUNK, CHUNK)]], rows_v.at[b], sem)

    def out_slice(ci):
      return out_hbm.at[pl.ds(base0 + ci * CHUNK, CHUNK)]

    gather(0, 0, gsems[0])

    @pl.loop(0, nchunks, step=2)
    def chunk_loop(ci0):
      for b in range(2):
        nb = 1 - b
        ci = ci0 + b

        @pl.when(ci + 1 < nchunks)
        def _issue_next():
          @pl.when(ci >= 1)
          def _drain_prev_out():
            pltpu.make_async_copy(rows_v.at[nb], out_slice(0), osems[nb]).wait()
          gather(ci + 1, nb, gsems[nb])

        pltpu.make_async_copy(
            tok_hbm.at[idx_v.at[pl.ds(0, CHUNK)]], rows_v.at[b],
            gsems[b]).wait()

        @pl.loop(0, MAXLEN, unroll=2)
        def t_body(t):
          pv0 = pos_v[t, pl.ds(0, LANES)]
          pv1 = pos_v[t, pl.ds(LANES, LANES)]
          for rep in range(REPS):
            r = rep * MAXLEN + t
            rows_v[b, r, pl.ds(0, LANES)] = rows_v[b, r, pl.ds(0, LANES)] + pv0
            rows_v[b, r, pl.ds(LANES, LANES)] = (
                rows_v[b, r, pl.ds(LANES, LANES)] + pv1)

        pltpu.async_copy(rows_v.at[b], out_slice(ci), osems[b])

    pltpu.make_async_copy(rows_v.at[0], out_slice(0), osems[nchunks % 2]).wait()
    pltpu.make_async_copy(rows_v.at[1], out_slice(0),
                          osems[(nchunks + 1) % 2]).wait()

  return emb_kernel


def kernel(x, token_table, pos_table):
  batch, maxlen = x.shape
  x_flat = x.reshape(-1).astype(jnp.int32)
  out = _make_kernel(x_flat.shape[0])(x_flat, token_table, pos_table)
  return out.reshape(batch, maxlen, EMBED)
